# baseline (speedup 1.0000x reference)
import functools

import numpy as np
import jax
import jax.numpy as jnp
from jax import lax
from jax.experimental import pallas as pl
from jax.experimental.pallas import tpu as pltpu
from jax.experimental.pallas import tpu_sc as plsc


def _cube_face_points_np(nx):
    faces = [
        (np.array([1.0, 0.0, 0.0]), np.array([0.0, 1.0, 0.0]), np.array([0.0, 0.0, 1.0])),
        (np.array([-1.0, 0.0, 0.0]), np.array([0.0, 0.0, 1.0]), np.array([0.0, 1.0, 0.0])),
        (np.array([0.0, 1.0, 0.0]), np.array([0.0, 0.0, 1.0]), np.array([1.0, 0.0, 0.0])),
        (np.array([0.0, -1.0, 0.0]), np.array([1.0, 0.0, 0.0]), np.array([0.0, 0.0, 1.0])),
        (np.array([0.0, 0.0, 1.0]), np.array([1.0, 0.0, 0.0]), np.array([0.0, 1.0, 0.0])),
        (np.array([0.0, 0.0, -1.0]), np.array([0.0, 1.0, 0.0]), np.array([1.0, 0.0, 0.0])),
    ]
    coords = np.linspace(-1.0 + 1.0 / nx, 1.0 - 1.0 / nx, nx)
    pts = []
    for n, u, v in faces:
        a, b = np.meshgrid(coords, coords, indexing='ij')
        p = n[None, None, :] + a[..., None] * u[None, None, :] + b[..., None] * v[None, None, :]
        p = p / np.linalg.norm(p, axis=-1, keepdims=True)
        pts.append(p.reshape(-1, 3))
    return np.concatenate(pts, 0)


@functools.lru_cache(maxsize=None)
def _graph_tables(nx, T, B):
    n_per = nx * nx
    N = T * n_per
    S = T * 4 * nx

    idx = np.arange(n_per).reshape(nx, nx)
    pairs = [
        (idx[:-1, :].ravel(), idx[1:, :].ravel()),
        (idx[:, :-1].ravel(), idx[:, 1:].ravel()),
    ]
    intra_src = np.concatenate([p[0] for p in pairs] + [p[1] for p in pairs])
    intra_dst = np.concatenate([p[1] for p in pairs] + [p[0] for p in pairs])

    pts = _cube_face_points_np(nx)
    bmask = np.zeros((nx, nx), bool)
    bmask[0, :] = True
    bmask[-1, :] = True
    bmask[:, 0] = True
    bmask[:, -1] = True
    bidx_tile = np.where(bmask.ravel())[0]
    bidx = np.concatenate([bidx_tile + t * n_per for t in range(T)])
    btile = np.repeat(np.arange(T), bidx_tile.size)
    bp = pts[bidx]
    d2 = ((bp[:, None, :] - bp[None, :, :]) ** 2).sum(-1)
    d2[btile[:, None] == btile[None, :]] = np.inf
    nn = np.argmin(d2, axis=1)

    cross_src = np.concatenate([bidx, bidx[nn]]).astype(np.int64)
    cross_dst = np.concatenate([bidx[nn], bidx]).astype(np.int64)

    dst_all = np.concatenate(
        [intra_dst + t * n_per for t in range(T)] + [cross_dst, np.arange(N)])
    deg = np.bincount(dst_all, minlength=N).astype(np.float64)
    norm = (1.0 / np.sqrt(np.maximum(deg, 1.0))).astype(np.float32)
    norm_grid = norm.reshape(T, nx, nx)

    tt = np.arange(N) // n_per
    rr = (np.arange(N) % n_per) // nx
    cc = np.arange(N) % nx
    slot = np.where(
        rr == 0, tt * 4 * nx + cc,
        np.where(rr == nx - 1, tt * 4 * nx + nx + cc,
                 np.where(cc == 0, tt * 4 * nx + 2 * nx + rr,
                          tt * 4 * nx + 3 * nx + rr)))

    nb = np.stack([norm_grid[:, 0, :], norm_grid[:, -1, :],
                   norm_grid[:, :, 0], norm_grid[:, :, -1]], axis=1)
    norm_bnd = np.tile(nb.reshape(S), B).reshape(B * S, 1)

    E = cross_src.size
    n_sub = 16
    per_sub = -(-E // n_sub)
    n_chunk = -(-per_sub // 128)
    pad_sub = n_chunk * 128
    src_tab = np.zeros((B, n_sub, n_chunk, 128), np.int32)
    dst_tab = np.full((B, n_sub, n_chunk, 128), S, np.int32)
    src_slot = slot[cross_src]
    dst_slot = slot[cross_dst]
    for b in range(B):
        for s in range(n_sub):
            lo = min(s * per_sub, E)
            hi = min(lo + per_sub, E)
            k = hi - lo
            flat_s = np.zeros(pad_sub, np.int32)
            flat_d = np.full(pad_sub, S, np.int32)
            flat_s[:k] = b * S + src_slot[lo:hi]
            flat_d[:k] = dst_slot[lo:hi]
            src_tab[b, s] = flat_s.reshape(n_chunk, 128)
            dst_tab[b, s] = flat_d.reshape(n_chunk, 128)

    return norm_grid, norm_bnd, src_tab, dst_tab, n_chunk


def _bnd_body(x_ref, w_ref, nb_ref, o_ref):
    h = jnp.dot(x_ref[...], w_ref[...], preferred_element_type=jnp.float32)
    o_ref[...] = h * nb_ref[...]


def _make_cross_kernel(B, S, F, n_chunk):
    n_sub = 16
    spm_rows = S + 128
    zrows = spm_rows // n_sub
    orows = S // n_sub
    mesh = plsc.VectorSubcoreMesh(core_axis_name="c", subcore_axis_name="s")

    @functools.partial(
        pl.kernel,
        mesh=mesh,
        out_type=jax.ShapeDtypeStruct((B, S, F), jnp.float32),
        scratch_types=[
            pltpu.VMEM((n_chunk, 128), jnp.int32),
            pltpu.VMEM((n_chunk, 128), jnp.int32),
            pltpu.VMEM((128, F), jnp.float32),
            pltpu.VMEM_SHARED((spm_rows, F), jnp.float32),
            pltpu.SemaphoreType.DMA,
        ],
    )
    def cross_kernel(gbnd, srci, dsti, zeros, out, vsrc, vdst, msgs, shared, sem):
        cid = lax.axis_index("c")
        sid = lax.axis_index("s")
        pltpu.sync_copy(srci.at[cid, sid], vsrc)
        pltpu.sync_copy(dsti.at[cid, sid], vdst)
        pltpu.sync_copy(zeros.at[pl.ds(sid * zrows, zrows)],
                        shared.at[pl.ds(sid * zrows, zrows)])
        plsc.subcore_barrier()
        for j in range(n_chunk):
            pltpu.async_copy(gbnd.at[vsrc.at[j]], msgs, sem).wait()
            pltpu.sync_copy(msgs, shared.at[vdst.at[j]], add=True)
        plsc.subcore_barrier()
        pltpu.sync_copy(shared.at[pl.ds(sid * orows, orows)],
                        out.at[cid, pl.ds(sid * orows, orows)])

    return cross_kernel


def _main_body(x_ref, w_ref, norm_ref, bias_ref, o_ref):
    nx, ny, F = x_ref.shape[2], x_ref.shape[3], x_ref.shape[4]
    x = x_ref[0, 0]
    h = jnp.dot(x.reshape(nx * ny, F), w_ref[...],
                preferred_element_type=jnp.float32)
    nrm = norm_ref[0]
    g = h.reshape(nx, ny, F) * nrm[:, :, None]
    zr = jnp.zeros((1, ny, F), jnp.float32)
    zc = jnp.zeros((nx, 1, F), jnp.float32)
    dn = jnp.concatenate([zr, g[:-1]], axis=0)
    up = jnp.concatenate([g[1:], zr], axis=0)
    rt = jnp.concatenate([zc, g[:, :-1]], axis=1)
    lf = jnp.concatenate([g[:, 1:], zc], axis=1)
    agg = g + up + dn + lf + rt
    o_ref[0, 0] = agg * nrm[:, :, None] + bias_ref[...][None, :, :]


def _patch_rows_body(o_in, c_ref, n_ref, o_ref):
    o_ref[...] = o_in[...] + c_ref[...] * n_ref[0, 0, 0][None, None, None, :, None]


def _patch_cols_body(o_in, c_ref, n_ref, o_ref):
    o_ref[...] = o_in[...] + c_ref[0, 0, 0] * n_ref[0, 0, 0][None, :, None]


def kernel(inputs, weight, bias):
    if len(inputs.shape) != 5:
        raise ValueError('inputs must be 5D')
    B, T, nx, ny, F = inputs.shape
    assert nx == ny and B == 2, "kernel specialized for B=2 square tiles"
    S = T * 4 * nx

    norm_grid_np, norm_bnd, src_tab, dst_tab, n_chunk = _graph_tables(nx, T, B)
    norm_rows = jnp.asarray(
        np.stack([norm_grid_np[:, 0, :], norm_grid_np[:, -1, :]],
                 axis=1).reshape(T, 2, 1, nx))
    norm_cols = jnp.asarray(
        np.stack([norm_grid_np[:, :, 0], norm_grid_np[:, :, -1]],
                 axis=1).reshape(T, 2, 1, nx))
    norm_grid = jnp.asarray(norm_grid_np)
    norm_bnd = jnp.asarray(norm_bnd)
    src_tab = jnp.asarray(src_tab)
    dst_tab = jnp.asarray(dst_tab)

    xbnd = jnp.stack([inputs[:, :, 0, :, :], inputs[:, :, nx - 1, :, :],
                      inputs[:, :, :, 0, :], inputs[:, :, :, ny - 1, :]],
                     axis=2)
    xbnd2d = xbnd.reshape(B * S, F)

    gbnd = pl.pallas_call(
        _bnd_body,
        out_shape=jax.ShapeDtypeStruct((B * S, F), jnp.float32),
    )(xbnd2d, weight, norm_bnd)

    zeros = jnp.zeros((S + 128, F), jnp.float32)
    cross = _make_cross_kernel(B, S, F, n_chunk)(gbnd, src_tab, dst_tab, zeros)
    cross = cross.reshape(B, T, 4, nx, F)

    out = pl.pallas_call(
        _main_body,
        grid=(B, T),
        in_specs=[
            pl.BlockSpec((1, 1, nx, ny, F), lambda b, t: (b, t, 0, 0, 0)),
            pl.BlockSpec((F, F), lambda b, t: (0, 0)),
            pl.BlockSpec((1, nx, ny), lambda b, t: (t, 0, 0)),
            pl.BlockSpec((1, F), lambda b, t: (0, 0)),
        ],
        out_specs=pl.BlockSpec((1, 1, nx, ny, F), lambda b, t: (b, t, 0, 0, 0)),
        out_shape=jax.ShapeDtypeStruct((B, T, nx, ny, F), jnp.float32),
    )(inputs, weight, norm_grid, bias.reshape(1, F))

    out = pl.pallas_call(
        _patch_rows_body,
        grid=(B, T, 2),
        in_specs=[
            pl.BlockSpec((1, 1, 1, ny, F), lambda b, t, s: (b, t, s * (nx - 1), 0, 0)),
            pl.BlockSpec((1, 1, 1, ny, F), lambda b, t, s: (b, t, s, 0, 0)),
            pl.BlockSpec((1, 1, 1, ny), lambda b, t, s: (t, s, 0, 0)),
        ],
        out_specs=pl.BlockSpec((1, 1, 1, ny, F), lambda b, t, s: (b, t, s * (nx - 1), 0, 0)),
        out_shape=jax.ShapeDtypeStruct((B, T, nx, ny, F), jnp.float32),
        input_output_aliases={0: 0},
    )(out, cross[:, :, 0:2], norm_rows)

    out4 = out.reshape(B, T, nx, ny * F)
    out4 = pl.pallas_call(
        _patch_cols_body,
        grid=(B, T, 2),
        in_specs=[
            pl.BlockSpec((1, 1, nx, F), lambda b, t, s: (b, t, 0, s * (ny - 1))),
            pl.BlockSpec((1, 1, 1, nx, F), lambda b, t, s: (b, t, s, 0, 0)),
            pl.BlockSpec((1, 1, 1, nx), lambda b, t, s: (t, s, 0, 0)),
        ],
        out_specs=pl.BlockSpec((1, 1, nx, F), lambda b, t, s: (b, t, 0, s * (ny - 1))),
        out_shape=jax.ShapeDtypeStruct((B, T, nx, ny * F), jnp.float32),
        input_output_aliases={0: 0},
    )(out4, cross[:, :, 2:4], norm_cols)
    return out4.reshape(B, T, nx, ny, F)


# speedup vs baseline: 2.1465x; 2.1465x over previous
import functools

import numpy as np
import jax
import jax.numpy as jnp
from jax import lax
from jax.experimental import pallas as pl
from jax.experimental.pallas import tpu as pltpu
from jax.experimental.pallas import tpu_sc as plsc


def _cube_face_points_np(nx):
    faces = [
        (np.array([1.0, 0.0, 0.0]), np.array([0.0, 1.0, 0.0]), np.array([0.0, 0.0, 1.0])),
        (np.array([-1.0, 0.0, 0.0]), np.array([0.0, 0.0, 1.0]), np.array([0.0, 1.0, 0.0])),
        (np.array([0.0, 1.0, 0.0]), np.array([0.0, 0.0, 1.0]), np.array([1.0, 0.0, 0.0])),
        (np.array([0.0, -1.0, 0.0]), np.array([1.0, 0.0, 0.0]), np.array([0.0, 0.0, 1.0])),
        (np.array([0.0, 0.0, 1.0]), np.array([1.0, 0.0, 0.0]), np.array([0.0, 1.0, 0.0])),
        (np.array([0.0, 0.0, -1.0]), np.array([0.0, 1.0, 0.0]), np.array([1.0, 0.0, 0.0])),
    ]
    coords = np.linspace(-1.0 + 1.0 / nx, 1.0 - 1.0 / nx, nx)
    pts = []
    for n, u, v in faces:
        a, b = np.meshgrid(coords, coords, indexing='ij')
        p = n[None, None, :] + a[..., None] * u[None, None, :] + b[..., None] * v[None, None, :]
        p = p / np.linalg.norm(p, axis=-1, keepdims=True)
        pts.append(p.reshape(-1, 3))
    return np.concatenate(pts, 0)


@functools.lru_cache(maxsize=None)
def _graph_tables(nx, T, B):
    n_per = nx * nx
    N = T * n_per
    S = T * 4 * nx

    idx = np.arange(n_per).reshape(nx, nx)
    pairs = [
        (idx[:-1, :].ravel(), idx[1:, :].ravel()),
        (idx[:, :-1].ravel(), idx[:, 1:].ravel()),
    ]
    intra_src = np.concatenate([p[0] for p in pairs] + [p[1] for p in pairs])
    intra_dst = np.concatenate([p[1] for p in pairs] + [p[0] for p in pairs])

    pts = _cube_face_points_np(nx)
    bmask = np.zeros((nx, nx), bool)
    bmask[0, :] = True
    bmask[-1, :] = True
    bmask[:, 0] = True
    bmask[:, -1] = True
    bidx_tile = np.where(bmask.ravel())[0]
    bidx = np.concatenate([bidx_tile + t * n_per for t in range(T)])
    btile = np.repeat(np.arange(T), bidx_tile.size)
    bp = pts[bidx]
    d2 = ((bp[:, None, :] - bp[None, :, :]) ** 2).sum(-1)
    d2[btile[:, None] == btile[None, :]] = np.inf
    nn = np.argmin(d2, axis=1)

    cross_src = np.concatenate([bidx, bidx[nn]]).astype(np.int64)
    cross_dst = np.concatenate([bidx[nn], bidx]).astype(np.int64)

    dst_all = np.concatenate(
        [intra_dst + t * n_per for t in range(T)] + [cross_dst, np.arange(N)])
    deg = np.bincount(dst_all, minlength=N).astype(np.float64)
    norm = (1.0 / np.sqrt(np.maximum(deg, 1.0))).astype(np.float32)
    norm_grid = norm.reshape(T, nx, nx)

    tt = np.arange(N) // n_per
    rr = (np.arange(N) % n_per) // nx
    cc = np.arange(N) % nx
    slot = np.where(
        rr == 0, tt * 4 * nx + cc,
        np.where(rr == nx - 1, tt * 4 * nx + nx + cc,
                 np.where(cc == 0, tt * 4 * nx + 2 * nx + rr,
                          tt * 4 * nx + 3 * nx + rr)))

    nb = np.stack([norm_grid[:, 0, :], norm_grid[:, -1, :],
                   norm_grid[:, :, 0], norm_grid[:, :, -1]], axis=1)
    norm_bnd = np.tile(nb.reshape(S), B).reshape(B * S, 1)

    E = cross_src.size
    n_sub = 16
    per_sub = -(-E // n_sub)
    n_chunk = -(-per_sub // 128)
    pad_sub = n_chunk * 128
    src_tab = np.zeros((B, n_sub, n_chunk, 128), np.int32)
    dst_tab = np.full((B, n_sub, n_chunk, 128), S, np.int32)
    src_slot = slot[cross_src]
    dst_slot = slot[cross_dst]
    for b in range(B):
        for s in range(n_sub):
            lo = min(s * per_sub, E)
            hi = min(lo + per_sub, E)
            k = hi - lo
            flat_s = np.zeros(pad_sub, np.int32)
            flat_d = np.full(pad_sub, S, np.int32)
            flat_s[:k] = b * S + src_slot[lo:hi]
            flat_d[:k] = dst_slot[lo:hi]
            src_tab[b, s] = flat_s.reshape(n_chunk, 128)
            dst_tab[b, s] = flat_d.reshape(n_chunk, 128)

    return norm_grid, norm_bnd, src_tab, dst_tab, n_chunk


def _bnd_body(x_ref, w_ref, nb_ref, o_ref):
    h = jnp.dot(x_ref[...], w_ref[...], preferred_element_type=jnp.float32)
    o_ref[...] = h * nb_ref[...]


def _make_cross_kernel(B, S, F, n_chunk):
    n_sub = 16
    spm_rows = S + 128
    zrows = spm_rows // n_sub
    orows = S // n_sub
    mesh = plsc.VectorSubcoreMesh(core_axis_name="c", subcore_axis_name="s")

    @functools.partial(
        pl.kernel,
        mesh=mesh,
        out_type=jax.ShapeDtypeStruct((B, S, F), jnp.float32),
        scratch_types=[
            pltpu.VMEM((n_chunk, 128), jnp.int32),
            pltpu.VMEM((n_chunk, 128), jnp.int32),
            pltpu.VMEM((128, F), jnp.float32),
            pltpu.VMEM_SHARED((spm_rows, F), jnp.float32),
            pltpu.SemaphoreType.DMA,
        ],
    )
    def cross_kernel(gbnd, srci, dsti, zeros, out, vsrc, vdst, msgs, shared, sem):
        cid = lax.axis_index("c")
        sid = lax.axis_index("s")
        pltpu.sync_copy(srci.at[cid, sid], vsrc)
        pltpu.sync_copy(dsti.at[cid, sid], vdst)
        pltpu.sync_copy(zeros.at[pl.ds(sid * zrows, zrows)],
                        shared.at[pl.ds(sid * zrows, zrows)])
        plsc.subcore_barrier()
        for j in range(n_chunk):
            pltpu.async_copy(gbnd.at[vsrc.at[j]], msgs, sem).wait()
            pltpu.sync_copy(msgs, shared.at[vdst.at[j]], add=True)
        plsc.subcore_barrier()
        pltpu.sync_copy(shared.at[pl.ds(sid * orows, orows)],
                        out.at[cid, pl.ds(sid * orows, orows)])

    return cross_kernel


def _main_body(x_ref, w_ref, norm_ref, bias_ref, o_ref):
    nx, ny, F = x_ref.shape[2], x_ref.shape[3], x_ref.shape[4]
    x = x_ref[0, 0]
    h = jnp.dot(x.reshape(nx * ny, F), w_ref[...],
                preferred_element_type=jnp.float32)
    nrm = norm_ref[0]
    g = h.reshape(nx, ny, F) * nrm[:, :, None]
    zr = jnp.zeros((1, ny, F), jnp.float32)
    zc = jnp.zeros((nx, 1, F), jnp.float32)
    dn = jnp.concatenate([zr, g[:-1]], axis=0)
    up = jnp.concatenate([g[1:], zr], axis=0)
    rt = jnp.concatenate([zc, g[:, :-1]], axis=1)
    lf = jnp.concatenate([g[:, 1:], zc], axis=1)
    agg = g + up + dn + lf + rt
    o_ref[0, 0] = agg * nrm[:, :, None] + bias_ref[...][None, :, :]


def _patch_rows_body(o_in, c_ref, n_ref, o_ref):
    o_ref[...] = o_in[...] + c_ref[...] * n_ref[0, 0, 0][None, None, None, :, None]


def _patch_cols_body(o_in, c_ref, n_ref, o_ref):
    s = pl.program_id(2)
    add = c_ref[0, 0, 0] * n_ref[0, 0, 0][:, None]
    col = jnp.where(s == 0, 0, 7)
    mask = (lax.broadcasted_iota(jnp.int32, (1, 1, 1, 8, 1), 3) == col)
    o_ref[...] = o_in[...] + jnp.where(mask, add[None, None, :, None, :], 0.0)


def kernel(inputs, weight, bias):
    if len(inputs.shape) != 5:
        raise ValueError('inputs must be 5D')
    B, T, nx, ny, F = inputs.shape
    assert nx == ny and B == 2, "kernel specialized for B=2 square tiles"
    S = T * 4 * nx

    norm_grid_np, norm_bnd, src_tab, dst_tab, n_chunk = _graph_tables(nx, T, B)
    norm_rows = jnp.asarray(
        np.stack([norm_grid_np[:, 0, :], norm_grid_np[:, -1, :]],
                 axis=1).reshape(T, 2, 1, nx))
    norm_cols = jnp.asarray(
        np.stack([norm_grid_np[:, :, 0], norm_grid_np[:, :, -1]],
                 axis=1).reshape(T, 2, 1, nx))
    norm_grid = jnp.asarray(norm_grid_np)
    norm_bnd = jnp.asarray(norm_bnd)
    src_tab = jnp.asarray(src_tab)
    dst_tab = jnp.asarray(dst_tab)

    xbnd = jnp.stack([inputs[:, :, 0, :, :], inputs[:, :, nx - 1, :, :],
                      inputs[:, :, :, 0, :], inputs[:, :, :, ny - 1, :]],
                     axis=2)
    xbnd2d = xbnd.reshape(B * S, F)

    gbnd = pl.pallas_call(
        _bnd_body,
        out_shape=jax.ShapeDtypeStruct((B * S, F), jnp.float32),
    )(xbnd2d, weight, norm_bnd)

    zeros = jnp.zeros((S + 128, F), jnp.float32)
    cross = _make_cross_kernel(B, S, F, n_chunk)(gbnd, src_tab, dst_tab, zeros)
    cross = cross.reshape(B, T, 4, nx, F)

    out = pl.pallas_call(
        _main_body,
        grid=(B, T),
        in_specs=[
            pl.BlockSpec((1, 1, nx, ny, F), lambda b, t: (b, t, 0, 0, 0)),
            pl.BlockSpec((F, F), lambda b, t: (0, 0)),
            pl.BlockSpec((1, nx, ny), lambda b, t: (t, 0, 0)),
            pl.BlockSpec((1, F), lambda b, t: (0, 0)),
        ],
        out_specs=pl.BlockSpec((1, 1, nx, ny, F), lambda b, t: (b, t, 0, 0, 0)),
        out_shape=jax.ShapeDtypeStruct((B, T, nx, ny, F), jnp.float32),
    )(inputs, weight, norm_grid, bias.reshape(1, F))

    out = pl.pallas_call(
        _patch_rows_body,
        grid=(B, T, 2),
        in_specs=[
            pl.BlockSpec((1, 1, 1, ny, F), lambda b, t, s: (b, t, s * (nx - 1), 0, 0)),
            pl.BlockSpec((1, 1, 1, ny, F), lambda b, t, s: (b, t, s, 0, 0)),
            pl.BlockSpec((1, 1, 1, ny), lambda b, t, s: (t, s, 0, 0)),
        ],
        out_specs=pl.BlockSpec((1, 1, 1, ny, F), lambda b, t, s: (b, t, s * (nx - 1), 0, 0)),
        out_shape=jax.ShapeDtypeStruct((B, T, nx, ny, F), jnp.float32),
        input_output_aliases={0: 0},
    )(out, cross[:, :, 0:2], norm_rows)

    out = pl.pallas_call(
        _patch_cols_body,
        grid=(B, T, 2),
        in_specs=[
            pl.BlockSpec((1, 1, nx, 8, F), lambda b, t, s: (b, t, 0, s * (ny // 8 - 1), 0)),
            pl.BlockSpec((1, 1, 1, nx, F), lambda b, t, s: (b, t, s, 0, 0)),
            pl.BlockSpec((1, 1, 1, nx), lambda b, t, s: (t, s, 0, 0)),
        ],
        out_specs=pl.BlockSpec((1, 1, nx, 8, F), lambda b, t, s: (b, t, 0, s * (ny // 8 - 1), 0)),
        out_shape=jax.ShapeDtypeStruct((B, T, nx, ny, F), jnp.float32),
        input_output_aliases={0: 0},
    )(out, cross[:, :, 2:4], norm_cols)
    return out


# speedup vs baseline: 2.7642x; 1.2878x over previous
import functools

import numpy as np
import jax
import jax.numpy as jnp
from jax import lax
from jax.experimental import pallas as pl
from jax.experimental.pallas import tpu as pltpu
from jax.experimental.pallas import tpu_sc as plsc


def _cube_face_points_np(nx):
    faces = [
        (np.array([1.0, 0.0, 0.0]), np.array([0.0, 1.0, 0.0]), np.array([0.0, 0.0, 1.0])),
        (np.array([-1.0, 0.0, 0.0]), np.array([0.0, 0.0, 1.0]), np.array([0.0, 1.0, 0.0])),
        (np.array([0.0, 1.0, 0.0]), np.array([0.0, 0.0, 1.0]), np.array([1.0, 0.0, 0.0])),
        (np.array([0.0, -1.0, 0.0]), np.array([1.0, 0.0, 0.0]), np.array([0.0, 0.0, 1.0])),
        (np.array([0.0, 0.0, 1.0]), np.array([1.0, 0.0, 0.0]), np.array([0.0, 1.0, 0.0])),
        (np.array([0.0, 0.0, -1.0]), np.array([0.0, 1.0, 0.0]), np.array([1.0, 0.0, 0.0])),
    ]
    coords = np.linspace(-1.0 + 1.0 / nx, 1.0 - 1.0 / nx, nx)
    pts = []
    for n, u, v in faces:
        a, b = np.meshgrid(coords, coords, indexing='ij')
        p = n[None, None, :] + a[..., None] * u[None, None, :] + b[..., None] * v[None, None, :]
        p = p / np.linalg.norm(p, axis=-1, keepdims=True)
        pts.append(p.reshape(-1, 3))
    return np.concatenate(pts, 0)


@functools.lru_cache(maxsize=None)
def _graph_tables(nx, T, B):
    n_per = nx * nx
    N = T * n_per
    S = T * 4 * nx

    idx = np.arange(n_per).reshape(nx, nx)
    pairs = [
        (idx[:-1, :].ravel(), idx[1:, :].ravel()),
        (idx[:, :-1].ravel(), idx[:, 1:].ravel()),
    ]
    intra_src = np.concatenate([p[0] for p in pairs] + [p[1] for p in pairs])
    intra_dst = np.concatenate([p[1] for p in pairs] + [p[0] for p in pairs])

    pts = _cube_face_points_np(nx)
    bmask = np.zeros((nx, nx), bool)
    bmask[0, :] = True
    bmask[-1, :] = True
    bmask[:, 0] = True
    bmask[:, -1] = True
    bidx_tile = np.where(bmask.ravel())[0]
    bidx = np.concatenate([bidx_tile + t * n_per for t in range(T)])
    btile = np.repeat(np.arange(T), bidx_tile.size)
    bp = pts[bidx]
    d2 = ((bp[:, None, :] - bp[None, :, :]) ** 2).sum(-1)
    d2[btile[:, None] == btile[None, :]] = np.inf
    nn = np.argmin(d2, axis=1)

    cross_src = np.concatenate([bidx, bidx[nn]]).astype(np.int64)
    cross_dst = np.concatenate([bidx[nn], bidx]).astype(np.int64)

    dst_all = np.concatenate(
        [intra_dst + t * n_per for t in range(T)] + [cross_dst, np.arange(N)])
    deg = np.bincount(dst_all, minlength=N).astype(np.float64)
    norm = (1.0 / np.sqrt(np.maximum(deg, 1.0))).astype(np.float32)
    norm_grid = norm.reshape(T, nx, nx)

    tt = np.arange(N) // n_per
    rr = (np.arange(N) % n_per) // nx
    cc = np.arange(N) % nx
    slot = np.where(
        rr == 0, tt * 4 * nx + cc,
        np.where(rr == nx - 1, tt * 4 * nx + nx + cc,
                 np.where(cc == 0, tt * 4 * nx + 2 * nx + rr,
                          tt * 4 * nx + 3 * nx + rr)))

    nb = np.stack([norm_grid[:, 0, :], norm_grid[:, -1, :],
                   norm_grid[:, :, 0], norm_grid[:, :, -1]], axis=1)
    norm_bnd = np.tile(nb.reshape(S), B).reshape(B * S, 1)

    E = cross_src.size
    n_sub = 16
    per_sub = -(-E // n_sub)
    n_chunk = -(-per_sub // 128)
    pad_sub = n_chunk * 128
    comb_tab = np.zeros((B, n_sub, 2, n_chunk, 128), np.int32)
    src_slot = slot[cross_src]
    dst_slot = slot[cross_dst]
    for b in range(B):
        for s in range(n_sub):
            lo = min(s * per_sub, E)
            hi = min(lo + per_sub, E)
            k = hi - lo
            flat_s = np.zeros(pad_sub, np.int32)
            flat_d = np.full(pad_sub, S, np.int32)
            flat_s[:k] = b * S + src_slot[lo:hi]
            flat_d[:k] = dst_slot[lo:hi]
            comb_tab[b, s, 0] = flat_s.reshape(n_chunk, 128)
            comb_tab[b, s, 1] = flat_d.reshape(n_chunk, 128)

    return norm_grid, norm_bnd, comb_tab, n_chunk


def _bnd_body(x_ref, w_ref, nb_ref, o_ref):
    h = jnp.dot(x_ref[...], w_ref[...], preferred_element_type=jnp.float32)
    o_ref[...] = h * nb_ref[...]


def _make_cross_kernel(B, S, F, n_chunk):
    n_sub = 16
    spm_rows = S + 128
    zrows = spm_rows // n_sub
    orows = S // n_sub
    mesh = plsc.VectorSubcoreMesh(core_axis_name="c", subcore_axis_name="s")

    @functools.partial(
        pl.kernel,
        mesh=mesh,
        out_type=jax.ShapeDtypeStruct((B, S, F), jnp.float32),
        scratch_types=[
            pltpu.VMEM((2, n_chunk, 128), jnp.int32),
            pltpu.VMEM((n_chunk, 128, F), jnp.float32),
            pltpu.VMEM_SHARED((spm_rows, F), jnp.float32),
        ] + [pltpu.SemaphoreType.DMA] * n_chunk,
    )
    def cross_kernel(gbnd, comb, zeros, out, vidx, msgs, shared, *sems):
        cid = lax.axis_index("c")
        sid = lax.axis_index("s")
        pltpu.sync_copy(comb.at[cid, sid], vidx)
        handles = [
            pltpu.async_copy(gbnd.at[vidx.at[0, j]], msgs.at[j], sems[j])
            for j in range(n_chunk)
        ]
        pltpu.sync_copy(zeros.at[pl.ds(sid * zrows, zrows)],
                        shared.at[pl.ds(sid * zrows, zrows)])
        plsc.subcore_barrier()
        for j in range(n_chunk):
            handles[j].wait()
            pltpu.sync_copy(msgs.at[j], shared.at[vidx.at[1, j]], add=True)
        plsc.subcore_barrier()
        pltpu.sync_copy(shared.at[pl.ds(sid * orows, orows)],
                        out.at[cid, pl.ds(sid * orows, orows)])

    return cross_kernel


def _main_body(x_ref, w_ref, norm_ref, cross_ref, bias_ref, o_ref):
    nx, ny, F = x_ref.shape[2], x_ref.shape[3], x_ref.shape[4]
    x = x_ref[0, 0]
    h = jnp.dot(x.reshape(nx * ny, F), w_ref[...],
                preferred_element_type=jnp.float32)
    nrm = norm_ref[0]
    g = h.reshape(nx, ny, F) * nrm[:, :, None]
    ca = cross_ref[0, 0]
    dn = jnp.concatenate([ca[0][None, :, :], g[:-1]], axis=0)
    up = jnp.concatenate([g[1:], ca[1][None, :, :]], axis=0)
    rt = jnp.concatenate([ca[2][:, None, :], g[:, :-1]], axis=1)
    lf = jnp.concatenate([g[:, 1:], ca[3][:, None, :]], axis=1)
    agg = g + up + dn + lf + rt
    o_ref[0, 0] = agg * nrm[:, :, None] + bias_ref[...][None, :, :]


def kernel(inputs, weight, bias):
    if len(inputs.shape) != 5:
        raise ValueError('inputs must be 5D')
    B, T, nx, ny, F = inputs.shape
    assert nx == ny and B == 2, "kernel specialized for B=2 square tiles"
    S = T * 4 * nx

    norm_grid_np, norm_bnd, comb_tab, n_chunk = _graph_tables(nx, T, B)
    norm_grid = jnp.asarray(norm_grid_np)
    norm_bnd = jnp.asarray(norm_bnd)
    comb_tab = jnp.asarray(comb_tab)

    xbnd = jnp.stack([inputs[:, :, 0, :, :], inputs[:, :, nx - 1, :, :],
                      inputs[:, :, :, 0, :], inputs[:, :, :, ny - 1, :]],
                     axis=2)
    xbnd2d = xbnd.reshape(B * S, F)

    gbnd = pl.pallas_call(
        _bnd_body,
        out_shape=jax.ShapeDtypeStruct((B * S, F), jnp.float32),
    )(xbnd2d, weight, norm_bnd)

    zeros = jnp.zeros((S + 128, F), jnp.float32)
    cross = _make_cross_kernel(B, S, F, n_chunk)(gbnd, comb_tab, zeros)
    cross = cross.reshape(B, T, 4, nx, F)

    out = pl.pallas_call(
        _main_body,
        grid=(B, T),
        in_specs=[
            pl.BlockSpec((1, 1, nx, ny, F), lambda b, t: (b, t, 0, 0, 0)),
            pl.BlockSpec((F, F), lambda b, t: (0, 0)),
            pl.BlockSpec((1, nx, ny), lambda b, t: (t, 0, 0)),
            pl.BlockSpec((1, 1, 4, nx, F), lambda b, t: (b, t, 0, 0, 0)),
            pl.BlockSpec((1, F), lambda b, t: (0, 0)),
        ],
        out_specs=pl.BlockSpec((1, 1, nx, ny, F), lambda b, t: (b, t, 0, 0, 0)),
        out_shape=jax.ShapeDtypeStruct((B, T, nx, ny, F), jnp.float32),
    )(inputs, weight, norm_grid, cross, bias.reshape(1, F))
    return out


# speedup vs baseline: 2.9019x; 1.0498x over previous
import functools

import numpy as np
import jax
import jax.numpy as jnp
from jax import lax
from jax.experimental import pallas as pl
from jax.experimental.pallas import tpu as pltpu
from jax.experimental.pallas import tpu_sc as plsc


def _cube_face_points_np(nx):
    faces = [
        (np.array([1.0, 0.0, 0.0]), np.array([0.0, 1.0, 0.0]), np.array([0.0, 0.0, 1.0])),
        (np.array([-1.0, 0.0, 0.0]), np.array([0.0, 0.0, 1.0]), np.array([0.0, 1.0, 0.0])),
        (np.array([0.0, 1.0, 0.0]), np.array([0.0, 0.0, 1.0]), np.array([1.0, 0.0, 0.0])),
        (np.array([0.0, -1.0, 0.0]), np.array([1.0, 0.0, 0.0]), np.array([0.0, 0.0, 1.0])),
        (np.array([0.0, 0.0, 1.0]), np.array([1.0, 0.0, 0.0]), np.array([0.0, 1.0, 0.0])),
        (np.array([0.0, 0.0, -1.0]), np.array([0.0, 1.0, 0.0]), np.array([1.0, 0.0, 0.0])),
    ]
    coords = np.linspace(-1.0 + 1.0 / nx, 1.0 - 1.0 / nx, nx)
    pts = []
    for n, u, v in faces:
        a, b = np.meshgrid(coords, coords, indexing='ij')
        p = n[None, None, :] + a[..., None] * u[None, None, :] + b[..., None] * v[None, None, :]
        p = p / np.linalg.norm(p, axis=-1, keepdims=True)
        pts.append(p.reshape(-1, 3))
    return np.concatenate(pts, 0)


@functools.lru_cache(maxsize=None)
def _graph_tables(nx, T, B):
    n_per = nx * nx
    N = T * n_per
    S = T * 4 * nx

    idx = np.arange(n_per).reshape(nx, nx)
    pairs = [
        (idx[:-1, :].ravel(), idx[1:, :].ravel()),
        (idx[:, :-1].ravel(), idx[:, 1:].ravel()),
    ]
    intra_src = np.concatenate([p[0] for p in pairs] + [p[1] for p in pairs])
    intra_dst = np.concatenate([p[1] for p in pairs] + [p[0] for p in pairs])

    pts = _cube_face_points_np(nx)
    bmask = np.zeros((nx, nx), bool)
    bmask[0, :] = True
    bmask[-1, :] = True
    bmask[:, 0] = True
    bmask[:, -1] = True
    bidx_tile = np.where(bmask.ravel())[0]
    bidx = np.concatenate([bidx_tile + t * n_per for t in range(T)])
    btile = np.repeat(np.arange(T), bidx_tile.size)
    bp = pts[bidx]
    d2 = ((bp[:, None, :] - bp[None, :, :]) ** 2).sum(-1)
    d2[btile[:, None] == btile[None, :]] = np.inf
    nn = np.argmin(d2, axis=1)

    cross_src = np.concatenate([bidx, bidx[nn]]).astype(np.int64)
    cross_dst = np.concatenate([bidx[nn], bidx]).astype(np.int64)

    dst_all = np.concatenate(
        [intra_dst + t * n_per for t in range(T)] + [cross_dst, np.arange(N)])
    deg = np.bincount(dst_all, minlength=N).astype(np.float64)
    norm = (1.0 / np.sqrt(np.maximum(deg, 1.0))).astype(np.float32)
    norm_grid = norm.reshape(T, nx, nx)

    tt = np.arange(N) // n_per
    rr = (np.arange(N) % n_per) // nx
    cc = np.arange(N) % nx
    slot = np.where(
        rr == 0, tt * 4 * nx + cc,
        np.where(rr == nx - 1, tt * 4 * nx + nx + cc,
                 np.where(cc == 0, tt * 4 * nx + 2 * nx + rr,
                          tt * 4 * nx + 3 * nx + rr)))

    nb = np.stack([norm_grid[:, 0, :], norm_grid[:, -1, :],
                   norm_grid[:, :, 0], norm_grid[:, :, -1]], axis=1)
    norm_bnd = np.tile(nb.reshape(S), B).reshape(B * S, 1)

    E = cross_src.size
    n_sub = 16
    per_sub = -(-E // n_sub)
    n_chunk = -(-per_sub // 128)
    pad_sub = n_chunk * 128
    comb_tab = np.zeros((B, n_sub, 2, n_chunk, 128), np.int32)
    dst_slot = slot[cross_dst]
    for b in range(B):
        for s in range(n_sub):
            lo = min(s * per_sub, E)
            hi = min(lo + per_sub, E)
            k = hi - lo
            flat_s = np.zeros(pad_sub, np.int32)
            flat_d = np.full(pad_sub, S, np.int32)
            flat_s[:k] = b * N + cross_src[lo:hi]
            flat_d[:k] = dst_slot[lo:hi]
            comb_tab[b, s, 0] = flat_s.reshape(n_chunk, 128)
            comb_tab[b, s, 1] = flat_d.reshape(n_chunk, 128)

    src_deg = deg[cross_src]
    dom_deg = np.bincount(src_deg.astype(np.int64)).argmax()
    src_norm = (1.0 / np.sqrt(np.maximum(src_deg, 1.0))).astype(np.float64)
    c_dom = float(1.0 / np.sqrt(dom_deg))
    odd = np.where(src_deg != dom_deg)[0]
    n_odd_pad = 32
    assert odd.size <= n_odd_pad
    e45_idx = np.zeros((B, n_odd_pad), np.int32)
    for b in range(B):
        e45_idx[b, :odd.size] = b * N + cross_src[odd]
    p45 = np.zeros((S, n_odd_pad), np.float32)
    for j, e in enumerate(odd):
        p45[dst_slot[e], j] += src_norm[e] - c_dom
    return norm_grid, comb_tab, n_chunk, e45_idx, p45, float(c_dom)


def _make_cross_g_body(c_dom):
    def body(acc_ref, e45_ref, p45_ref, w_ref, o_ref):
        corr = jnp.dot(p45_ref[...], e45_ref[0],
                       preferred_element_type=jnp.float32)
        crossx = acc_ref[0] * c_dom + corr
        o_ref[0] = jnp.dot(crossx, w_ref[...],
                           preferred_element_type=jnp.float32)
    return body


def _make_cross_kernel(B, S, F, n_chunk):
    n_sub = 16
    spm_rows = S + 128
    zrows = spm_rows // n_sub
    orows = S // n_sub
    mesh = plsc.VectorSubcoreMesh(core_axis_name="c", subcore_axis_name="s")

    @functools.partial(
        pl.kernel,
        mesh=mesh,
        out_type=[jax.ShapeDtypeStruct((B, S, F), jnp.float32),
                  jax.ShapeDtypeStruct((B, 32, F), jnp.float32)],
        scratch_types=[
            pltpu.VMEM((2, n_chunk, 128), jnp.int32),
            pltpu.VMEM((n_chunk, 128, F), jnp.float32),
            pltpu.VMEM((32,), jnp.int32),
            pltpu.VMEM((32, F), jnp.float32),
            pltpu.VMEM_SHARED((spm_rows, F), jnp.float32),
        ] + [pltpu.SemaphoreType.DMA] * (n_chunk + 1),
    )
    def cross_kernel(xrows, comb, e45i, zeros, out, out45,
                     vidx, msgs, vi45, m45, shared, *sems):
        cid = lax.axis_index("c")
        sid = lax.axis_index("s")
        pltpu.sync_copy(comb.at[cid, sid], vidx)
        handles = [
            pltpu.async_copy(xrows.at[vidx.at[0, j]], msgs.at[j], sems[j])
            for j in range(n_chunk)
        ]
        pltpu.sync_copy(zeros.at[pl.ds(sid * zrows, zrows)],
                        shared.at[pl.ds(sid * zrows, zrows)])
        @pl.when(sid == 0)
        def _():
            pltpu.sync_copy(e45i.at[cid], vi45)
            pltpu.async_copy(xrows.at[vi45], m45, sems[n_chunk]).wait()
            pltpu.sync_copy(m45, out45.at[cid])
        plsc.subcore_barrier()
        for j in range(n_chunk):
            handles[j].wait()
            pltpu.sync_copy(msgs.at[j], shared.at[vidx.at[1, j]], add=True)
        plsc.subcore_barrier()
        pltpu.sync_copy(shared.at[pl.ds(sid * orows, orows)],
                        out.at[cid, pl.ds(sid * orows, orows)])

    return cross_kernel


def _main_body(x_ref, w_ref, norm_ref, cross_ref, bias_ref, o_ref):
    nx, ny, F = x_ref.shape[2], x_ref.shape[3], x_ref.shape[4]
    x = x_ref[0, 0]
    h = jnp.dot(x.reshape(nx * ny, F), w_ref[...],
                preferred_element_type=jnp.float32)
    nrm = norm_ref[0]
    g = h.reshape(nx, ny, F) * nrm[:, :, None]
    ca = cross_ref[0, 0]
    dn = jnp.concatenate([ca[0][None, :, :], g[:-1]], axis=0)
    up = jnp.concatenate([g[1:], ca[1][None, :, :]], axis=0)
    rt = jnp.concatenate([ca[2][:, None, :], g[:, :-1]], axis=1)
    lf = jnp.concatenate([g[:, 1:], ca[3][:, None, :]], axis=1)
    agg = g + up + dn + lf + rt
    o_ref[0, 0] = agg * nrm[:, :, None] + bias_ref[...][None, :, :]


def kernel(inputs, weight, bias):
    if len(inputs.shape) != 5:
        raise ValueError('inputs must be 5D')
    B, T, nx, ny, F = inputs.shape
    assert nx == ny and B == 2, "kernel specialized for B=2 square tiles"
    S = T * 4 * nx

    norm_grid_np, comb_tab, n_chunk, e45_idx, p45, c_dom = _graph_tables(nx, T, B)
    norm_grid = jnp.asarray(norm_grid_np)
    comb_tab = jnp.asarray(comb_tab)
    e45_idx = jnp.asarray(e45_idx)
    p45 = jnp.asarray(p45)

    xrows = inputs.reshape(B * T * nx * ny, F)
    zeros = jnp.zeros((S + 128, F), jnp.float32)
    acc, e45 = _make_cross_kernel(B, S, F, n_chunk)(
        xrows, comb_tab, e45_idx, zeros)

    cross = pl.pallas_call(
        _make_cross_g_body(c_dom),
        grid=(B,),
        in_specs=[
            pl.BlockSpec((1, S, F), lambda b: (b, 0, 0)),
            pl.BlockSpec((1, 32, F), lambda b: (b, 0, 0)),
            pl.BlockSpec((S, 32), lambda b: (0, 0)),
            pl.BlockSpec((F, F), lambda b: (0, 0)),
        ],
        out_specs=pl.BlockSpec((1, S, F), lambda b: (b, 0, 0)),
        out_shape=jax.ShapeDtypeStruct((B, S, F), jnp.float32),
    )(acc, e45, p45, weight)
    cross = cross.reshape(B, T, 4, nx, F)

    out = pl.pallas_call(
        _main_body,
        grid=(B, T),
        in_specs=[
            pl.BlockSpec((1, 1, nx, ny, F), lambda b, t: (b, t, 0, 0, 0)),
            pl.BlockSpec((F, F), lambda b, t: (0, 0)),
            pl.BlockSpec((1, nx, ny), lambda b, t: (t, 0, 0)),
            pl.BlockSpec((1, 1, 4, nx, F), lambda b, t: (b, t, 0, 0, 0)),
            pl.BlockSpec((1, F), lambda b, t: (0, 0)),
        ],
        out_specs=pl.BlockSpec((1, 1, nx, ny, F), lambda b, t: (b, t, 0, 0, 0)),
        out_shape=jax.ShapeDtypeStruct((B, T, nx, ny, F), jnp.float32),
    )(inputs, weight, norm_grid, cross, bias.reshape(1, F))
    return out


# speedup vs baseline: 2.9944x; 1.0319x over previous
import functools

import numpy as np
import jax
import jax.numpy as jnp
from jax import lax
from jax.experimental import pallas as pl
from jax.experimental.pallas import tpu as pltpu
from jax.experimental.pallas import tpu_sc as plsc


def _cube_face_points_np(nx):
    faces = [
        (np.array([1.0, 0.0, 0.0]), np.array([0.0, 1.0, 0.0]), np.array([0.0, 0.0, 1.0])),
        (np.array([-1.0, 0.0, 0.0]), np.array([0.0, 0.0, 1.0]), np.array([0.0, 1.0, 0.0])),
        (np.array([0.0, 1.0, 0.0]), np.array([0.0, 0.0, 1.0]), np.array([1.0, 0.0, 0.0])),
        (np.array([0.0, -1.0, 0.0]), np.array([1.0, 0.0, 0.0]), np.array([0.0, 0.0, 1.0])),
        (np.array([0.0, 0.0, 1.0]), np.array([1.0, 0.0, 0.0]), np.array([0.0, 1.0, 0.0])),
        (np.array([0.0, 0.0, -1.0]), np.array([0.0, 1.0, 0.0]), np.array([1.0, 0.0, 0.0])),
    ]
    coords = np.linspace(-1.0 + 1.0 / nx, 1.0 - 1.0 / nx, nx)
    pts = []
    for n, u, v in faces:
        a, b = np.meshgrid(coords, coords, indexing='ij')
        p = n[None, None, :] + a[..., None] * u[None, None, :] + b[..., None] * v[None, None, :]
        p = p / np.linalg.norm(p, axis=-1, keepdims=True)
        pts.append(p.reshape(-1, 3))
    return np.concatenate(pts, 0)


@functools.lru_cache(maxsize=None)
def _graph_tables(nx, T, B):
    n_per = nx * nx
    N = T * n_per
    S = T * 4 * nx

    idx = np.arange(n_per).reshape(nx, nx)
    pairs = [
        (idx[:-1, :].ravel(), idx[1:, :].ravel()),
        (idx[:, :-1].ravel(), idx[:, 1:].ravel()),
    ]
    intra_src = np.concatenate([p[0] for p in pairs] + [p[1] for p in pairs])
    intra_dst = np.concatenate([p[1] for p in pairs] + [p[0] for p in pairs])

    pts = _cube_face_points_np(nx)
    bmask = np.zeros((nx, nx), bool)
    bmask[0, :] = True
    bmask[-1, :] = True
    bmask[:, 0] = True
    bmask[:, -1] = True
    bidx_tile = np.where(bmask.ravel())[0]
    bidx = np.concatenate([bidx_tile + t * n_per for t in range(T)])
    btile = np.repeat(np.arange(T), bidx_tile.size)
    bp = pts[bidx]
    d2 = ((bp[:, None, :] - bp[None, :, :]) ** 2).sum(-1)
    d2[btile[:, None] == btile[None, :]] = np.inf
    nn = np.argmin(d2, axis=1)

    cross_src = np.concatenate([bidx, bidx[nn]]).astype(np.int64)
    cross_dst = np.concatenate([bidx[nn], bidx]).astype(np.int64)

    dst_all = np.concatenate(
        [intra_dst + t * n_per for t in range(T)] + [cross_dst, np.arange(N)])
    deg = np.bincount(dst_all, minlength=N).astype(np.float64)
    norm = (1.0 / np.sqrt(np.maximum(deg, 1.0))).astype(np.float32)
    norm_grid = norm.reshape(T, nx, nx)

    tt = np.arange(N) // n_per
    rr = (np.arange(N) % n_per) // nx
    cc = np.arange(N) % nx
    slot = np.where(
        rr == 0, tt * 4 * nx + cc,
        np.where(rr == nx - 1, tt * 4 * nx + nx + cc,
                 np.where(cc == 0, tt * 4 * nx + 2 * nx + rr,
                          tt * 4 * nx + 3 * nx + rr)))

    nb = np.stack([norm_grid[:, 0, :], norm_grid[:, -1, :],
                   norm_grid[:, :, 0], norm_grid[:, :, -1]], axis=1)
    norm_bnd = np.tile(nb.reshape(S), B).reshape(B * S, 1)

    E = cross_src.size
    n_sub = 16
    per_sub = -(-E // n_sub)
    n_chunk = -(-per_sub // 128)
    pad_sub = n_chunk * 128
    comb_tab = np.zeros((B, n_sub, 2, n_chunk, 128), np.int32)
    dst_slot = slot[cross_dst]
    for b in range(B):
        for s in range(n_sub):
            lo = min(s * per_sub, E)
            hi = min(lo + per_sub, E)
            k = hi - lo
            flat_s = np.zeros(pad_sub, np.int32)
            flat_d = np.full(pad_sub, S, np.int32)
            flat_s[:k] = b * N + cross_src[lo:hi]
            flat_d[:k] = dst_slot[lo:hi]
            comb_tab[b, s, 0] = flat_s.reshape(n_chunk, 128)
            comb_tab[b, s, 1] = flat_d.reshape(n_chunk, 128)

    src_deg = deg[cross_src]
    dom_deg = np.bincount(src_deg.astype(np.int64)).argmax()
    src_norm = (1.0 / np.sqrt(np.maximum(src_deg, 1.0))).astype(np.float64)
    c_dom = float(1.0 / np.sqrt(dom_deg))
    odd = np.where(src_deg != dom_deg)[0]
    n_odd_pad = 32
    assert odd.size <= n_odd_pad
    e45_idx = np.zeros((B, n_odd_pad), np.int32)
    for b in range(B):
        e45_idx[b, :odd.size] = b * N + cross_src[odd]
    p45 = np.zeros((S, n_odd_pad), np.float32)
    for j, e in enumerate(odd):
        p45[dst_slot[e], j] += src_norm[e] - c_dom
    return norm_grid, comb_tab, n_chunk, e45_idx, p45, float(c_dom)


def _make_cross_kernel(B, S, F, n_chunk):
    n_sub = 16
    spm_rows = S + 128
    zrows = spm_rows // n_sub
    orows = S // n_sub
    mesh = plsc.VectorSubcoreMesh(core_axis_name="c", subcore_axis_name="s")

    @functools.partial(
        pl.kernel,
        mesh=mesh,
        out_type=[jax.ShapeDtypeStruct((B, S, F), jnp.float32),
                  jax.ShapeDtypeStruct((B, 32, F), jnp.float32)],
        scratch_types=[
            pltpu.VMEM((2, n_chunk, 128), jnp.int32),
            pltpu.VMEM((n_chunk, 128, F), jnp.float32),
            pltpu.VMEM((32,), jnp.int32),
            pltpu.VMEM((32, F), jnp.float32),
            pltpu.VMEM_SHARED((spm_rows, F), jnp.float32),
        ] + [pltpu.SemaphoreType.DMA] * (n_chunk + 1),
    )
    def cross_kernel(xrows, comb, e45i, zeros, out, out45,
                     vidx, msgs, vi45, m45, shared, *sems):
        cid = lax.axis_index("c")
        sid = lax.axis_index("s")
        pltpu.sync_copy(comb.at[cid, sid], vidx)
        handles = [
            pltpu.async_copy(xrows.at[vidx.at[0, j]], msgs.at[j], sems[j])
            for j in range(n_chunk)
        ]
        pltpu.sync_copy(zeros.at[pl.ds(sid * zrows, zrows)],
                        shared.at[pl.ds(sid * zrows, zrows)])
        @pl.when(sid == 0)
        def _():
            pltpu.sync_copy(e45i.at[cid], vi45)
            pltpu.async_copy(xrows.at[vi45], m45, sems[n_chunk]).wait()
            pltpu.sync_copy(m45, out45.at[cid])
        plsc.subcore_barrier()
        for j in range(n_chunk):
            handles[j].wait()
            pltpu.sync_copy(msgs.at[j], shared.at[vidx.at[1, j]], add=True)
        plsc.subcore_barrier()
        pltpu.sync_copy(shared.at[pl.ds(sid * orows, orows)],
                        out.at[cid, pl.ds(sid * orows, orows)])

    return cross_kernel


def _make_main_body(c_dom):
    def _main_body(x_ref, w_ref, norm_ref, acc_ref, p45_ref, e45_ref,
                   bias_ref, o_ref):
        nx, ny, F = x_ref.shape[2], x_ref.shape[3], x_ref.shape[4]
        x = x_ref[0, 0]
        h = jnp.dot(x.reshape(nx * ny, F), w_ref[...],
                    preferred_element_type=jnp.float32)
        nrm = norm_ref[0]
        g = h.reshape(nx, ny, F) * nrm[:, :, None]
        crossx = acc_ref[0] * c_dom + jnp.dot(
            p45_ref[...], e45_ref[0], preferred_element_type=jnp.float32)
        ca = jnp.dot(crossx, w_ref[...],
                     preferred_element_type=jnp.float32).reshape(4, nx, F)
        dn = jnp.concatenate([ca[0][None, :, :], g[:-1]], axis=0)
        up = jnp.concatenate([g[1:], ca[1][None, :, :]], axis=0)
        rt = jnp.concatenate([ca[2][:, None, :], g[:, :-1]], axis=1)
        lf = jnp.concatenate([g[:, 1:], ca[3][:, None, :]], axis=1)
        agg = g + up + dn + lf + rt
        o_ref[0, 0] = agg * nrm[:, :, None] + bias_ref[...][None, :, :]
    return _main_body


def kernel(inputs, weight, bias):
    if len(inputs.shape) != 5:
        raise ValueError('inputs must be 5D')
    B, T, nx, ny, F = inputs.shape
    assert nx == ny and B == 2, "kernel specialized for B=2 square tiles"
    S = T * 4 * nx

    norm_grid_np, comb_tab, n_chunk, e45_idx, p45, c_dom = _graph_tables(nx, T, B)
    norm_grid = jnp.asarray(norm_grid_np)
    comb_tab = jnp.asarray(comb_tab)
    e45_idx = jnp.asarray(e45_idx)
    p45 = jnp.asarray(p45)

    xrows = inputs.reshape(B * T * nx * ny, F)
    zeros = jnp.zeros((S + 128, F), jnp.float32)
    acc, e45 = _make_cross_kernel(B, S, F, n_chunk)(
        xrows, comb_tab, e45_idx, zeros)

    out = pl.pallas_call(
        _make_main_body(c_dom),
        grid=(B, T),
        in_specs=[
            pl.BlockSpec((1, 1, nx, ny, F), lambda b, t: (b, t, 0, 0, 0)),
            pl.BlockSpec((F, F), lambda b, t: (0, 0)),
            pl.BlockSpec((1, nx, ny), lambda b, t: (t, 0, 0)),
            pl.BlockSpec((1, 4 * nx, F), lambda b, t: (b, t, 0)),
            pl.BlockSpec((4 * nx, 32), lambda b, t: (t, 0)),
            pl.BlockSpec((1, 32, F), lambda b, t: (b, 0, 0)),
            pl.BlockSpec((1, F), lambda b, t: (0, 0)),
        ],
        out_specs=pl.BlockSpec((1, 1, nx, ny, F), lambda b, t: (b, t, 0, 0, 0)),
        out_shape=jax.ShapeDtypeStruct((B, T, nx, ny, F), jnp.float32),
    )(inputs, weight, norm_grid, acc, p45, e45, bias.reshape(1, F))
    return out


# speedup vs baseline: 3.0025x; 1.0027x over previous
import functools

import numpy as np
import jax
import jax.numpy as jnp
from jax import lax
from jax.experimental import pallas as pl
from jax.experimental.pallas import tpu as pltpu
from jax.experimental.pallas import tpu_sc as plsc


def _cube_face_points_np(nx):
    faces = [
        (np.array([1.0, 0.0, 0.0]), np.array([0.0, 1.0, 0.0]), np.array([0.0, 0.0, 1.0])),
        (np.array([-1.0, 0.0, 0.0]), np.array([0.0, 0.0, 1.0]), np.array([0.0, 1.0, 0.0])),
        (np.array([0.0, 1.0, 0.0]), np.array([0.0, 0.0, 1.0]), np.array([1.0, 0.0, 0.0])),
        (np.array([0.0, -1.0, 0.0]), np.array([1.0, 0.0, 0.0]), np.array([0.0, 0.0, 1.0])),
        (np.array([0.0, 0.0, 1.0]), np.array([1.0, 0.0, 0.0]), np.array([0.0, 1.0, 0.0])),
        (np.array([0.0, 0.0, -1.0]), np.array([0.0, 1.0, 0.0]), np.array([1.0, 0.0, 0.0])),
    ]
    coords = np.linspace(-1.0 + 1.0 / nx, 1.0 - 1.0 / nx, nx)
    pts = []
    for n, u, v in faces:
        a, b = np.meshgrid(coords, coords, indexing='ij')
        p = n[None, None, :] + a[..., None] * u[None, None, :] + b[..., None] * v[None, None, :]
        p = p / np.linalg.norm(p, axis=-1, keepdims=True)
        pts.append(p.reshape(-1, 3))
    return np.concatenate(pts, 0)


@functools.lru_cache(maxsize=None)
def _graph_tables(nx, T, B):
    n_per = nx * nx
    N = T * n_per
    S = T * 4 * nx

    idx = np.arange(n_per).reshape(nx, nx)
    pairs = [
        (idx[:-1, :].ravel(), idx[1:, :].ravel()),
        (idx[:, :-1].ravel(), idx[:, 1:].ravel()),
    ]
    intra_src = np.concatenate([p[0] for p in pairs] + [p[1] for p in pairs])
    intra_dst = np.concatenate([p[1] for p in pairs] + [p[0] for p in pairs])

    pts = _cube_face_points_np(nx)
    bmask = np.zeros((nx, nx), bool)
    bmask[0, :] = True
    bmask[-1, :] = True
    bmask[:, 0] = True
    bmask[:, -1] = True
    bidx_tile = np.where(bmask.ravel())[0]
    bidx = np.concatenate([bidx_tile + t * n_per for t in range(T)])
    btile = np.repeat(np.arange(T), bidx_tile.size)
    bp = pts[bidx]
    d2 = ((bp[:, None, :] - bp[None, :, :]) ** 2).sum(-1)
    d2[btile[:, None] == btile[None, :]] = np.inf
    nn = np.argmin(d2, axis=1)

    cross_src = np.concatenate([bidx, bidx[nn]]).astype(np.int64)
    cross_dst = np.concatenate([bidx[nn], bidx]).astype(np.int64)

    dst_all = np.concatenate(
        [intra_dst + t * n_per for t in range(T)] + [cross_dst, np.arange(N)])
    deg = np.bincount(dst_all, minlength=N).astype(np.float64)
    norm = (1.0 / np.sqrt(np.maximum(deg, 1.0))).astype(np.float32)
    norm_grid = norm.reshape(T, nx, nx)

    tt = np.arange(N) // n_per
    rr = (np.arange(N) % n_per) // nx
    cc = np.arange(N) % nx
    slot = np.where(
        rr == 0, tt * 4 * nx + cc,
        np.where(rr == nx - 1, tt * 4 * nx + nx + cc,
                 np.where(cc == 0, tt * 4 * nx + 2 * nx + rr,
                          tt * 4 * nx + 3 * nx + rr)))

    nb = np.stack([norm_grid[:, 0, :], norm_grid[:, -1, :],
                   norm_grid[:, :, 0], norm_grid[:, :, -1]], axis=1)
    norm_bnd = np.tile(nb.reshape(S), B).reshape(B * S, 1)

    E = cross_src.size
    n_sub = 16
    per_sub = -(-E // n_sub)
    n_chunk = -(-per_sub // 128)
    pad_sub = n_chunk * 128
    comb_tab = np.zeros((B, n_sub, 2, n_chunk, 128), np.int32)
    dst_slot = slot[cross_dst]
    for b in range(B):
        for s in range(n_sub):
            lo = min(s * per_sub, E)
            hi = min(lo + per_sub, E)
            k = hi - lo
            flat_s = np.zeros(pad_sub, np.int32)
            flat_d = np.full(pad_sub, S, np.int32)
            flat_s[:k] = b * N + cross_src[lo:hi]
            flat_d[:k] = dst_slot[lo:hi]
            comb_tab[b, s, 0] = flat_s.reshape(n_chunk, 128)
            comb_tab[b, s, 1] = flat_d.reshape(n_chunk, 128)

    src_deg = deg[cross_src]
    dom_deg = np.bincount(src_deg.astype(np.int64)).argmax()
    src_norm = (1.0 / np.sqrt(np.maximum(src_deg, 1.0))).astype(np.float64)
    c_dom = float(1.0 / np.sqrt(dom_deg))
    odd = np.where(src_deg != dom_deg)[0]
    n_odd_pad = 32
    assert odd.size <= n_odd_pad
    e45_idx = np.zeros((B, n_odd_pad), np.int32)
    for b in range(B):
        e45_idx[b, :odd.size] = b * N + cross_src[odd]
    p45 = np.zeros((S, n_odd_pad), np.float32)
    for j, e in enumerate(odd):
        p45[dst_slot[e], j] += src_norm[e] - c_dom
    return norm_grid, comb_tab, n_chunk, e45_idx, p45, float(c_dom)


def _make_cross_kernel(B, S, F, n_chunk):
    n_sub = 16
    spm_rows = S + 128
    zrows = spm_rows // n_sub
    orows = S // n_sub
    mesh = plsc.VectorSubcoreMesh(core_axis_name="c", subcore_axis_name="s")

    @functools.partial(
        pl.kernel,
        mesh=mesh,
        out_type=[jax.ShapeDtypeStruct((B, S, F), jnp.float32),
                  jax.ShapeDtypeStruct((B, 32, F), jnp.float32)],
        scratch_types=[
            pltpu.VMEM((2, n_chunk, 128), jnp.int32),
            pltpu.VMEM((n_chunk, 128, F), jnp.float32),
            pltpu.VMEM((32,), jnp.int32),
            pltpu.VMEM((32, F), jnp.float32),
            pltpu.VMEM_SHARED((spm_rows, F), jnp.float32),
        ] + [pltpu.SemaphoreType.DMA] * (n_chunk + 1),
    )
    def cross_kernel(xrows, comb, e45i, zeros, out, out45,
                     vidx, msgs, vi45, m45, shared, *sems):
        cid = lax.axis_index("c")
        sid = lax.axis_index("s")
        pltpu.sync_copy(comb.at[cid, sid], vidx)
        handles = [
            pltpu.async_copy(xrows.at[vidx.at[0, j]], msgs.at[j], sems[j])
            for j in range(n_chunk)
        ]
        pltpu.sync_copy(zeros.at[pl.ds(sid * zrows, zrows)],
                        shared.at[pl.ds(sid * zrows, zrows)])
        @pl.when(sid == 0)
        def _():
            pltpu.sync_copy(e45i.at[cid], vi45)
            pltpu.async_copy(xrows.at[vi45], m45, sems[n_chunk]).wait()
            pltpu.sync_copy(m45, out45.at[cid])
        plsc.subcore_barrier()
        for j in range(n_chunk):
            handles[j].wait()
            pltpu.sync_copy(msgs.at[j], shared.at[vidx.at[1, j]], add=True)
        plsc.subcore_barrier()
        pltpu.sync_copy(shared.at[pl.ds(sid * orows, orows)],
                        out.at[cid, pl.ds(sid * orows, orows)])

    return cross_kernel


def _make_main_body(c_dom):
    def _main_body(x_ref, w_ref, norm_ref, acc_ref, p45_ref, e45_ref,
                   bias_ref, o_ref):
        nx, ny, F = x_ref.shape[2], x_ref.shape[3], x_ref.shape[4]
        x = x_ref[0, 0]
        h = jnp.dot(x.reshape(nx * ny, F), w_ref[...],
                    preferred_element_type=jnp.float32)
        nrm = norm_ref[0]
        g = h.reshape(nx, ny, F) * nrm[:, :, None]
        crossx = acc_ref[0] * c_dom + jnp.dot(
            p45_ref[...], e45_ref[0], preferred_element_type=jnp.float32)
        ca = jnp.dot(crossx, w_ref[...],
                     preferred_element_type=jnp.float32).reshape(4, nx, F)
        dn = jnp.concatenate([ca[0][None, :, :], g[:-1]], axis=0)
        up = jnp.concatenate([g[1:], ca[1][None, :, :]], axis=0)
        rt = jnp.concatenate([ca[2][:, None, :], g[:, :-1]], axis=1)
        lf = jnp.concatenate([g[:, 1:], ca[3][:, None, :]], axis=1)
        agg = g + up + dn + lf + rt
        o_ref[0, 0] = agg * nrm[:, :, None] + bias_ref[...][None, :, :]
    return _main_body


def kernel(inputs, weight, bias):
    if len(inputs.shape) != 5:
        raise ValueError('inputs must be 5D')
    B, T, nx, ny, F = inputs.shape
    assert nx == ny and B == 2, "kernel specialized for B=2 square tiles"
    S = T * 4 * nx

    norm_grid_np, comb_tab, n_chunk, e45_idx, p45, c_dom = _graph_tables(nx, T, B)
    norm_grid = jnp.asarray(norm_grid_np)
    comb_tab = jnp.asarray(comb_tab)
    e45_idx = jnp.asarray(e45_idx)
    p45 = jnp.asarray(p45)

    xrows = inputs.reshape(B * T * nx * ny, F)
    zeros = jnp.zeros((S + 128, F), jnp.float32)
    acc, e45 = _make_cross_kernel(B, S, F, n_chunk)(
        xrows, comb_tab, e45_idx, zeros)

    out = pl.pallas_call(
        _make_main_body(c_dom),
        grid=(B, T),
        in_specs=[
            pl.BlockSpec((1, 1, nx, ny, F), lambda b, t: (b, t, 0, 0, 0)),
            pl.BlockSpec((F, F), lambda b, t: (0, 0)),
            pl.BlockSpec((1, nx, ny), lambda b, t: (t, 0, 0)),
            pl.BlockSpec((1, 4 * nx, F), lambda b, t: (b, t, 0)),
            pl.BlockSpec((4 * nx, 32), lambda b, t: (t, 0)),
            pl.BlockSpec((1, 32, F), lambda b, t: (b, 0, 0)),
            pl.BlockSpec((1, F), lambda b, t: (0, 0)),
        ],
        out_specs=pl.BlockSpec((1, 1, nx, ny, F), lambda b, t: (b, t, 0, 0, 0)),
        out_shape=jax.ShapeDtypeStruct((B, T, nx, ny, F), jnp.float32),
        compiler_params=pltpu.CompilerParams(
            dimension_semantics=("parallel", "parallel")),
    )(inputs, weight, norm_grid, acc, p45, e45, bias.reshape(1, F))
    return out
